# baseline (speedup 1.0000x reference)
import functools

import jax
import jax.numpy as jnp
from jax import lax
from jax.experimental import pallas as pl
from jax.experimental.pallas import tpu as pltpu
from jax.experimental.pallas import tpu_sc as plsc

N = 10000
E = 320000
D = 128
T = 100
B = 100
G4 = 512

NC = 2
NS = 16
K = 128
NCHUNK = 80
EPT = NCHUNK * K
EPC = EPT * NS
E_PAD = EPC * NC
N_PAD = 10240
RPT = N_PAD // NS
ZR = 128

_mesh = functools.partial(
    plsc.VectorSubcoreMesh,
    core_axis_name="c", subcore_axis_name="s", num_cores=NC, num_subcores=NS,
)


def _deg_body(dst_hbm, out_hbm, dst_v, ones_v, acc_sh, sem):
    cid = lax.axis_index("c")
    sid = lax.axis_index("s")
    rowbase = cid * (EPC // K) + sid * NCHUNK
    pltpu.sync_copy(dst_hbm.at[pl.ds(rowbase, NCHUNK)], dst_v)

    z16 = jnp.zeros((16,), jnp.float32)
    o16 = jnp.full((16,), 1.0, jnp.float32)

    @pl.loop(0, K)
    def _(r):
        for j in range(D // 16):
            ones_v[r, pl.ds(j * 16, 16)] = z16

    for t in range(RPT // K):
        pltpu.sync_copy(ones_v, acc_sh.at[pl.ds(sid * RPT + t * K, K)])

    @pl.loop(0, K)
    def _(r):
        for j in range(D // 16):
            ones_v[r, pl.ds(j * 16, 16)] = o16

    plsc.subcore_barrier()

    @pl.loop(0, NCHUNK)
    def _(j):
        pltpu.sync_copy(ones_v, acc_sh.at[dst_v.at[j]], add=True)

    plsc.subcore_barrier()
    pltpu.sync_copy(acc_sh.at[pl.ds(sid * RPT, RPT)],
                    out_hbm.at[cid, pl.ds(sid * RPT, RPT)])


_deg_call = pl.kernel(
    _deg_body,
    out_type=jax.ShapeDtypeStruct((NC, N_PAD, D), jnp.float32),
    mesh=_mesh(),
    scratch_types=[
        pltpu.VMEM((NCHUNK, K), jnp.int32),
        pltpu.VMEM((K, D), jnp.float32),
        pltpu.VMEM_SHARED((N_PAD, D), jnp.float32),
        pltpu.SemaphoreType.DMA,
    ],
)


HC = 40
T_SLOW = 1 * HC
T_FAST = 3 * HC
SLOW_CID = 0


def _edge_body(hn_hbm, src_hbm, dst_hbm, out_hbm,
               src_v, dst_v, rows0_v, rows1_v, acc_sh, sem0, sem1):
    cid = lax.axis_index("c")
    sid = lax.axis_index("s")
    is_slow = cid == SLOW_CID
    tpc = jnp.where(is_slow, T_SLOW, T_FAST)
    core_base = jnp.where(is_slow, 0, NS * T_SLOW)
    base = core_base + sid * tpc
    nph = jnp.where(is_slow, T_SLOW // HC, T_FAST // HC)

    z16 = jnp.zeros((16,), jnp.float32)

    @pl.loop(0, K)
    def _(r):
        for j in range(D // 16):
            rows0_v[r, pl.ds(j * 16, 16)] = z16

    for t in range(RPT // K):
        pltpu.sync_copy(rows0_v, acc_sh.at[pl.ds(sid * RPT + t * K, K)])
    plsc.subcore_barrier()

    for p in range(T_FAST // HC):
        @pl.when(p < nph)
        def _():
            pltpu.sync_copy(src_hbm.at[pl.ds(base + p * HC, HC)], src_v)
            pltpu.sync_copy(dst_hbm.at[pl.ds(base + p * HC, HC)], dst_v)
            pltpu.async_copy(hn_hbm.at[src_v.at[0]], rows0_v, sem0)

            @pl.loop(0, HC, step=2)
            def _(j):
                pltpu.async_copy(hn_hbm.at[src_v.at[j + 1]], rows1_v, sem1)
                pltpu.make_async_copy(
                    hn_hbm.at[src_v.at[j]], rows0_v, sem0).wait()
                pltpu.sync_copy(rows0_v, acc_sh.at[dst_v.at[j]], add=True)

                @pl.when(j + 2 < HC)
                def _():
                    pltpu.async_copy(hn_hbm.at[src_v.at[j + 2]], rows0_v, sem0)

                pltpu.make_async_copy(
                    hn_hbm.at[src_v.at[j + 1]], rows1_v, sem1).wait()
                pltpu.sync_copy(rows1_v, acc_sh.at[dst_v.at[j + 1]], add=True)

    plsc.subcore_barrier()
    pltpu.sync_copy(acc_sh.at[pl.ds(sid * RPT, RPT)],
                    out_hbm.at[cid, pl.ds(sid * RPT, RPT)])


_edge_call = pl.kernel(
    _edge_body,
    out_type=jax.ShapeDtypeStruct((NC, N_PAD, D), jnp.float32),
    mesh=_mesh(),
    scratch_types=[
        pltpu.VMEM((HC, K), jnp.int32),
        pltpu.VMEM((HC, K), jnp.int32),
        pltpu.VMEM((K, D), jnp.float32),
        pltpu.VMEM((K, D), jnp.float32),
        pltpu.VMEM_SHARED((N_PAD, D), jnp.float32),
        pltpu.SemaphoreType.DMA,
        pltpu.SemaphoreType.DMA,
    ],
)


R = 2000


def _scale_mm_body(x_ref, w_ref, degp_ref, hn_ref, dinv_ref):
    deg = degp_ref[0, :, 0:1] + degp_ref[1, :, 0:1] + 1.0
    dinv = lax.rsqrt(deg)
    hn_ref[...] = jnp.dot(x_ref[...], w_ref[...],
                          preferred_element_type=jnp.float32) * dinv
    dinv_ref[...] = jnp.broadcast_to(dinv, dinv_ref.shape)


_scale_mm = pl.pallas_call(
    _scale_mm_body,
    grid=(N // R,),
    in_specs=[
        pl.BlockSpec((R, D), lambda i: (i, 0)),
        pl.BlockSpec((D, D), lambda i: (0, 0)),
        pl.BlockSpec((2, R, D), lambda i: (0, i, 0)),
    ],
    out_specs=[
        pl.BlockSpec((R, D), lambda i: (i, 0)),
        pl.BlockSpec((R, 16), lambda i: (i, 0)),
    ],
    out_shape=[
        jax.ShapeDtypeStruct((N, D), jnp.float32),
        jax.ShapeDtypeStruct((N, 16), jnp.float32),
    ],
)


def _mid_body(acc_ref, hn_ref, dinv_ref, b_ref, w_ref, out_ref):
    dinv = dinv_ref[:, 0:1]
    h = jnp.maximum(
        (acc_ref[0] + acc_ref[1] + hn_ref[...]) * dinv + b_ref[...], 0.0)
    out_ref[...] = jnp.dot(h, w_ref[...],
                           preferred_element_type=jnp.float32) * dinv


_mid_call = pl.pallas_call(
    _mid_body,
    grid=(N // R,),
    in_specs=[
        pl.BlockSpec((2, R, D), lambda i: (0, i, 0)),
        pl.BlockSpec((R, D), lambda i: (i, 0)),
        pl.BlockSpec((R, 16), lambda i: (i, 0)),
        pl.BlockSpec((1, D), lambda i: (0, 0)),
        pl.BlockSpec((D, D), lambda i: (0, 0)),
    ],
    out_specs=pl.BlockSpec((R, D), lambda i: (i, 0)),
    out_shape=jax.ShapeDtypeStruct((N, D), jnp.float32),
)


def _last_body(acc_ref, hn_ref, dinv_ref, b_ref, w_ref, ob_ref, out_ref):
    dinv = dinv_ref[:, 0:1]
    h = jnp.maximum(
        (acc_ref[0] + acc_ref[1] + hn_ref[...]) * dinv + b_ref[...], 0.0)
    out_ref[...] = jnp.dot(h, w_ref[...],
                           preferred_element_type=jnp.float32) + ob_ref[...]


_last_call = pl.pallas_call(
    _last_body,
    grid=(N // R,),
    in_specs=[
        pl.BlockSpec((2, R, D), lambda i: (0, i, 0)),
        pl.BlockSpec((R, D), lambda i: (i, 0)),
        pl.BlockSpec((R, 16), lambda i: (i, 0)),
        pl.BlockSpec((1, D), lambda i: (0, 0)),
        pl.BlockSpec((D, G4), lambda i: (0, 0)),
        pl.BlockSpec((1, G4), lambda i: (0, 0)),
    ],
    out_specs=pl.BlockSpec((R, G4), lambda i: (i, 0)),
    out_shape=jax.ShapeDtypeStruct((N, G4), jnp.float32),
)


def _lstm_body(xp_ref, whh_ref, wp_ref, bp_ref, out_ref, h_s, c_s):
    t = pl.program_id(0)

    @pl.when(t == 0)
    def _():
        h_s[...] = jnp.zeros_like(h_s)
        c_s[...] = jnp.zeros_like(c_s)

    gates = xp_ref[0] + jnp.dot(h_s[...], whh_ref[...],
                                      preferred_element_type=jnp.float32)
    i = jax.nn.sigmoid(gates[:, 0:D])
    f = jax.nn.sigmoid(gates[:, D:2 * D])
    g = jnp.tanh(gates[:, 2 * D:3 * D])
    o = jax.nn.sigmoid(gates[:, 3 * D:4 * D])
    c = f * c_s[...] + i * g
    h = o * jnp.tanh(c)
    h_s[...] = h
    c_s[...] = c

    @pl.when(t == T - 1)
    def _():
        out_ref[...] = jnp.dot(h, wp_ref[...],
                               preferred_element_type=jnp.float32) + bp_ref[...]


_lstm_call = pl.pallas_call(
    _lstm_body,
    grid=(T,),
    in_specs=[
        pl.BlockSpec((1, B, G4), lambda t: (t, 0, 0)),
        pl.BlockSpec((D, G4), lambda t: (0, 0)),
        pl.BlockSpec((D, D), lambda t: (0, 0)),
        pl.BlockSpec((1, D), lambda t: (0, 0)),
    ],
    out_specs=pl.BlockSpec((B, D), lambda t: (0, 0)),
    out_shape=jax.ShapeDtypeStruct((B, D), jnp.float32),
    scratch_shapes=[
        pltpu.VMEM((B, D), jnp.float32),
        pltpu.VMEM((B, D), jnp.float32),
    ],
)


def kernel(x, edge_index, batch_size, W1, b1, W2, b2,
           W_ih, W_hh, b_ih, b_hh, Wp, bp):
    pad = E_PAD - E
    src = jnp.concatenate([edge_index[0], jnp.zeros((pad,), jnp.int32)])
    dst = jnp.concatenate([edge_index[1], jnp.full((pad,), N, jnp.int32)])
    srcg = src.reshape(E_PAD // K, K)
    dstg = dst.reshape(E_PAD // K, K)

    degp = _deg_call(dstg)
    hn1, dinv = _scale_mm(x, W1, degp)

    acc1 = _edge_call(hn1, srcg, dstg)
    hn2 = _mid_call(acc1, hn1, dinv, b1.reshape(1, D), W2)
    acc2 = _edge_call(hn2, srcg, dstg)
    xproj = _last_call(acc2, hn2, dinv, b2.reshape(1, D), W_ih.T,
                       (b_ih + b_hh).reshape(1, G4))
    xtm = xproj.reshape(B, T, G4).transpose(1, 0, 2)
    out = _lstm_call(xtm, W_hh.T, Wp.T, bp.reshape(1, D))
    return out


# speedup vs baseline: 1.0125x; 1.0125x over previous
import functools

import jax
import jax.numpy as jnp
from jax import lax
from jax.experimental import pallas as pl
from jax.experimental.pallas import tpu as pltpu
from jax.experimental.pallas import tpu_sc as plsc

N = 10000
E = 320000
D = 128
T = 100
B = 100
G4 = 512

NC = 2
NS = 16
K = 128
NCHUNK = 80
EPT = NCHUNK * K
EPC = EPT * NS
E_PAD = EPC * NC
N_PAD = 10240
RPT = N_PAD // NS
ZR = 128

_mesh = functools.partial(
    plsc.VectorSubcoreMesh,
    core_axis_name="c", subcore_axis_name="s", num_cores=NC, num_subcores=NS,
)


def _deg_body(dst_hbm, out_hbm, dst_v, ones_v, acc_sh, sem):
    cid = lax.axis_index("c")
    sid = lax.axis_index("s")
    rowbase = cid * (EPC // K) + sid * NCHUNK
    pltpu.sync_copy(dst_hbm.at[pl.ds(rowbase, NCHUNK)], dst_v)

    z16 = jnp.zeros((16,), jnp.float32)
    o16 = jnp.full((16,), 1.0, jnp.float32)

    @pl.loop(0, K)
    def _(r):
        for j in range(D // 16):
            ones_v[r, pl.ds(j * 16, 16)] = z16

    for t in range(RPT // K):
        pltpu.sync_copy(ones_v, acc_sh.at[pl.ds(sid * RPT + t * K, K)])

    @pl.loop(0, K)
    def _(r):
        for j in range(D // 16):
            ones_v[r, pl.ds(j * 16, 16)] = o16

    plsc.subcore_barrier()

    @pl.loop(0, NCHUNK)
    def _(j):
        pltpu.sync_copy(ones_v, acc_sh.at[dst_v.at[j]], add=True)

    plsc.subcore_barrier()
    pltpu.sync_copy(acc_sh.at[pl.ds(sid * RPT, RPT)],
                    out_hbm.at[cid, pl.ds(sid * RPT, RPT)])


_deg_call = pl.kernel(
    _deg_body,
    out_type=jax.ShapeDtypeStruct((NC, N_PAD, D), jnp.float32),
    mesh=_mesh(),
    scratch_types=[
        pltpu.VMEM((NCHUNK, K), jnp.int32),
        pltpu.VMEM((K, D), jnp.float32),
        pltpu.VMEM_SHARED((N_PAD, D), jnp.float32),
        pltpu.SemaphoreType.DMA,
    ],
)


HC = 40
T_SLOW = 1 * HC
T_FAST = 3 * HC
SLOW_CID = 1


def _edge_body(hn_hbm, src_hbm, dst_hbm, out_hbm,
               src_v, dst_v, rows0_v, rows1_v, acc_sh, sem0, sem1):
    cid = lax.axis_index("c")
    sid = lax.axis_index("s")
    is_slow = cid == SLOW_CID
    tpc = jnp.where(is_slow, T_SLOW, T_FAST)
    core_base = jnp.where(is_slow, 0, NS * T_SLOW)
    base = core_base + sid * tpc
    nph = jnp.where(is_slow, T_SLOW // HC, T_FAST // HC)

    z16 = jnp.zeros((16,), jnp.float32)

    @pl.loop(0, K)
    def _(r):
        for j in range(D // 16):
            rows0_v[r, pl.ds(j * 16, 16)] = z16

    for t in range(RPT // K):
        pltpu.sync_copy(rows0_v, acc_sh.at[pl.ds(sid * RPT + t * K, K)])
    plsc.subcore_barrier()

    for p in range(T_FAST // HC):
        @pl.when(p < nph)
        def _():
            pltpu.sync_copy(src_hbm.at[pl.ds(base + p * HC, HC)], src_v)
            pltpu.sync_copy(dst_hbm.at[pl.ds(base + p * HC, HC)], dst_v)
            pltpu.async_copy(hn_hbm.at[src_v.at[0]], rows0_v, sem0)

            @pl.loop(0, HC, step=2)
            def _(j):
                pltpu.async_copy(hn_hbm.at[src_v.at[j + 1]], rows1_v, sem1)
                pltpu.make_async_copy(
                    hn_hbm.at[src_v.at[j]], rows0_v, sem0).wait()
                pltpu.sync_copy(rows0_v, acc_sh.at[dst_v.at[j]], add=True)

                @pl.when(j + 2 < HC)
                def _():
                    pltpu.async_copy(hn_hbm.at[src_v.at[j + 2]], rows0_v, sem0)

                pltpu.make_async_copy(
                    hn_hbm.at[src_v.at[j + 1]], rows1_v, sem1).wait()
                pltpu.sync_copy(rows1_v, acc_sh.at[dst_v.at[j + 1]], add=True)

    plsc.subcore_barrier()
    pltpu.sync_copy(acc_sh.at[pl.ds(sid * RPT, RPT)],
                    out_hbm.at[cid, pl.ds(sid * RPT, RPT)])


_edge_call = pl.kernel(
    _edge_body,
    out_type=jax.ShapeDtypeStruct((NC, N_PAD, D), jnp.float32),
    mesh=_mesh(),
    scratch_types=[
        pltpu.VMEM((HC, K), jnp.int32),
        pltpu.VMEM((HC, K), jnp.int32),
        pltpu.VMEM((K, D), jnp.float32),
        pltpu.VMEM((K, D), jnp.float32),
        pltpu.VMEM_SHARED((N_PAD, D), jnp.float32),
        pltpu.SemaphoreType.DMA,
        pltpu.SemaphoreType.DMA,
    ],
)


R = 2000


def _scale_mm_body(x_ref, w_ref, degp_ref, hn_ref, dinv_ref):
    deg = degp_ref[0, :, 0:1] + degp_ref[1, :, 0:1] + 1.0
    dinv = lax.rsqrt(deg)
    hn_ref[...] = jnp.dot(x_ref[...], w_ref[...],
                          preferred_element_type=jnp.float32) * dinv
    dinv_ref[...] = jnp.broadcast_to(dinv, dinv_ref.shape)


_scale_mm = pl.pallas_call(
    _scale_mm_body,
    grid=(N // R,),
    in_specs=[
        pl.BlockSpec((R, D), lambda i: (i, 0)),
        pl.BlockSpec((D, D), lambda i: (0, 0)),
        pl.BlockSpec((2, R, D), lambda i: (0, i, 0)),
    ],
    out_specs=[
        pl.BlockSpec((R, D), lambda i: (i, 0)),
        pl.BlockSpec((R, 16), lambda i: (i, 0)),
    ],
    out_shape=[
        jax.ShapeDtypeStruct((N, D), jnp.float32),
        jax.ShapeDtypeStruct((N, 16), jnp.float32),
    ],
)


def _mid_body(acc_ref, hn_ref, dinv_ref, b_ref, w_ref, out_ref):
    dinv = dinv_ref[:, 0:1]
    h = jnp.maximum(
        (acc_ref[0] + acc_ref[1] + hn_ref[...]) * dinv + b_ref[...], 0.0)
    out_ref[...] = jnp.dot(h, w_ref[...],
                           preferred_element_type=jnp.float32) * dinv


_mid_call = pl.pallas_call(
    _mid_body,
    grid=(N // R,),
    in_specs=[
        pl.BlockSpec((2, R, D), lambda i: (0, i, 0)),
        pl.BlockSpec((R, D), lambda i: (i, 0)),
        pl.BlockSpec((R, 16), lambda i: (i, 0)),
        pl.BlockSpec((1, D), lambda i: (0, 0)),
        pl.BlockSpec((D, D), lambda i: (0, 0)),
    ],
    out_specs=pl.BlockSpec((R, D), lambda i: (i, 0)),
    out_shape=jax.ShapeDtypeStruct((N, D), jnp.float32),
)


def _last_body(acc_ref, hn_ref, dinv_ref, b_ref, w_ref, ob_ref, out_ref):
    dinv = dinv_ref[:, 0:1]
    h = jnp.maximum(
        (acc_ref[0] + acc_ref[1] + hn_ref[...]) * dinv + b_ref[...], 0.0)
    out_ref[...] = jnp.dot(h, w_ref[...],
                           preferred_element_type=jnp.float32) + ob_ref[...]


_last_call = pl.pallas_call(
    _last_body,
    grid=(N // R,),
    in_specs=[
        pl.BlockSpec((2, R, D), lambda i: (0, i, 0)),
        pl.BlockSpec((R, D), lambda i: (i, 0)),
        pl.BlockSpec((R, 16), lambda i: (i, 0)),
        pl.BlockSpec((1, D), lambda i: (0, 0)),
        pl.BlockSpec((D, G4), lambda i: (0, 0)),
        pl.BlockSpec((1, G4), lambda i: (0, 0)),
    ],
    out_specs=pl.BlockSpec((R, G4), lambda i: (i, 0)),
    out_shape=jax.ShapeDtypeStruct((N, G4), jnp.float32),
)


def _lstm_body(xp_ref, whh_ref, wp_ref, bp_ref, out_ref, h_s, c_s):
    t = pl.program_id(0)

    @pl.when(t == 0)
    def _():
        h_s[...] = jnp.zeros_like(h_s)
        c_s[...] = jnp.zeros_like(c_s)

    gates = xp_ref[0] + jnp.dot(h_s[...], whh_ref[...],
                                      preferred_element_type=jnp.float32)
    i = jax.nn.sigmoid(gates[:, 0:D])
    f = jax.nn.sigmoid(gates[:, D:2 * D])
    g = jnp.tanh(gates[:, 2 * D:3 * D])
    o = jax.nn.sigmoid(gates[:, 3 * D:4 * D])
    c = f * c_s[...] + i * g
    h = o * jnp.tanh(c)
    h_s[...] = h
    c_s[...] = c

    @pl.when(t == T - 1)
    def _():
        out_ref[...] = jnp.dot(h, wp_ref[...],
                               preferred_element_type=jnp.float32) + bp_ref[...]


_lstm_call = pl.pallas_call(
    _lstm_body,
    grid=(T,),
    in_specs=[
        pl.BlockSpec((1, B, G4), lambda t: (t, 0, 0)),
        pl.BlockSpec((D, G4), lambda t: (0, 0)),
        pl.BlockSpec((D, D), lambda t: (0, 0)),
        pl.BlockSpec((1, D), lambda t: (0, 0)),
    ],
    out_specs=pl.BlockSpec((B, D), lambda t: (0, 0)),
    out_shape=jax.ShapeDtypeStruct((B, D), jnp.float32),
    scratch_shapes=[
        pltpu.VMEM((B, D), jnp.float32),
        pltpu.VMEM((B, D), jnp.float32),
    ],
)


def kernel(x, edge_index, batch_size, W1, b1, W2, b2,
           W_ih, W_hh, b_ih, b_hh, Wp, bp):
    pad = E_PAD - E
    src = jnp.concatenate([edge_index[0], jnp.zeros((pad,), jnp.int32)])
    dst = jnp.concatenate([edge_index[1], jnp.full((pad,), N, jnp.int32)])
    srcg = src.reshape(E_PAD // K, K)
    dstg = dst.reshape(E_PAD // K, K)

    degp = _deg_call(dstg)
    hn1, dinv = _scale_mm(x, W1, degp)

    acc1 = _edge_call(hn1, srcg, dstg)
    hn2 = _mid_call(acc1, hn1, dinv, b1.reshape(1, D), W2)
    acc2 = _edge_call(hn2, srcg, dstg)
    xproj = _last_call(acc2, hn2, dinv, b2.reshape(1, D), W_ih.T,
                       (b_ih + b_hh).reshape(1, G4))
    xtm = xproj.reshape(B, T, G4).transpose(1, 0, 2)
    out = _lstm_call(xtm, W_hh.T, Wp.T, bp.reshape(1, D))
    return out


# speedup vs baseline: 2.3510x; 2.3220x over previous
import functools

import jax
import jax.numpy as jnp
from jax import lax
from jax.experimental import pallas as pl
from jax.experimental.pallas import tpu as pltpu
from jax.experimental.pallas import tpu_sc as plsc

N = 10000
E = 320000
D = 128
T = 100
B = 100
G4 = 512

NC = 2
NS = 16
K = 128
NCHUNK = 80
EPT = NCHUNK * K
EPC = EPT * NS
E_PAD = EPC * NC
N_PAD = 10240
RPT = N_PAD // NS
ZR = 128

_mesh = functools.partial(
    plsc.VectorSubcoreMesh,
    core_axis_name="c", subcore_axis_name="s", num_cores=NC, num_subcores=NS,
)


def _deg_body(dst_hbm, out_hbm, dst_v, ones_v, acc_sh, sem):
    cid = lax.axis_index("c")
    sid = lax.axis_index("s")
    rowbase = cid * (EPC // K) + sid * NCHUNK
    pltpu.sync_copy(dst_hbm.at[pl.ds(rowbase, NCHUNK)], dst_v)

    z16 = jnp.zeros((16,), jnp.float32)
    o16 = jnp.full((16,), 1.0, jnp.float32)

    @pl.loop(0, K)
    def _(r):
        for j in range(D // 16):
            ones_v[r, pl.ds(j * 16, 16)] = z16

    for t in range(RPT // K):
        pltpu.sync_copy(ones_v, acc_sh.at[pl.ds(sid * RPT + t * K, K)])

    @pl.loop(0, K)
    def _(r):
        for j in range(D // 16):
            ones_v[r, pl.ds(j * 16, 16)] = o16

    plsc.subcore_barrier()

    @pl.loop(0, NCHUNK)
    def _(j):
        pltpu.sync_copy(ones_v, acc_sh.at[dst_v.at[j]], add=True)

    plsc.subcore_barrier()
    pltpu.sync_copy(acc_sh.at[pl.ds(sid * RPT, RPT)],
                    out_hbm.at[cid, pl.ds(sid * RPT, RPT)])


_deg_call = pl.kernel(
    _deg_body,
    out_type=jax.ShapeDtypeStruct((NC, N_PAD, D), jnp.float32),
    mesh=_mesh(),
    scratch_types=[
        pltpu.VMEM((NCHUNK, K), jnp.int32),
        pltpu.VMEM((K, D), jnp.float32),
        pltpu.VMEM_SHARED((N_PAD, D), jnp.float32),
        pltpu.SemaphoreType.DMA,
    ],
)


HC = 40
T_SLOW = 2 * HC
T_FAST = 2 * HC
SLOW_CID = 1


def _edge_body(hn_hbm, src_hbm, dst_hbm, out_hbm,
               src_v, dst_v, rows0_v, rows1_v, acc_sh, sem0, sem1):
    cid = lax.axis_index("c")
    sid = lax.axis_index("s")
    is_slow = cid == SLOW_CID
    tpc = jnp.where(is_slow, T_SLOW, T_FAST)
    core_base = jnp.where(is_slow, 0, NS * T_SLOW)
    base = core_base + sid * tpc
    nph = jnp.where(is_slow, T_SLOW // HC, T_FAST // HC)

    z16 = jnp.zeros((16,), jnp.float32)

    @pl.loop(0, K)
    def _(r):
        for j in range(D // 16):
            rows0_v[r, pl.ds(j * 16, 16)] = z16

    for t in range(RPT // K):
        pltpu.sync_copy(rows0_v, acc_sh.at[pl.ds(sid * RPT + t * K, K)])
    plsc.subcore_barrier()

    for p in range(T_FAST // HC):
        @pl.when(p < nph)
        def _():
            pltpu.sync_copy(src_hbm.at[pl.ds(base + p * HC, HC)], src_v)
            pltpu.sync_copy(dst_hbm.at[pl.ds(base + p * HC, HC)], dst_v)
            pltpu.async_copy(hn_hbm.at[src_v.at[0]], rows0_v, sem0)

            @pl.loop(0, HC, step=2)
            def _(j):
                pltpu.async_copy(hn_hbm.at[src_v.at[j + 1]], rows1_v, sem1)
                pltpu.make_async_copy(
                    hn_hbm.at[src_v.at[j]], rows0_v, sem0).wait()
                pltpu.sync_copy(rows0_v, acc_sh.at[dst_v.at[j]], add=True)

                @pl.when(j + 2 < HC)
                def _():
                    pltpu.async_copy(hn_hbm.at[src_v.at[j + 2]], rows0_v, sem0)

                pltpu.make_async_copy(
                    hn_hbm.at[src_v.at[j + 1]], rows1_v, sem1).wait()
                pltpu.sync_copy(rows1_v, acc_sh.at[dst_v.at[j + 1]], add=True)

    plsc.subcore_barrier()
    pltpu.sync_copy(acc_sh.at[pl.ds(sid * RPT, RPT)],
                    out_hbm.at[cid, pl.ds(sid * RPT, RPT)])


_edge_call = pl.kernel(
    _edge_body,
    out_type=jax.ShapeDtypeStruct((NC, N_PAD, D), jnp.float32),
    mesh=_mesh(),
    scratch_types=[
        pltpu.VMEM((HC, K), jnp.int32),
        pltpu.VMEM((HC, K), jnp.int32),
        pltpu.VMEM((K, D), jnp.float32),
        pltpu.VMEM((K, D), jnp.float32),
        pltpu.VMEM_SHARED((N_PAD, D), jnp.float32),
        pltpu.SemaphoreType.DMA,
        pltpu.SemaphoreType.DMA,
    ],
)


R = 2000


def _scale_mm_body(x_ref, w_ref, degp_ref, hn_ref, dinv_ref):
    deg = degp_ref[0, :, 0:1] + degp_ref[1, :, 0:1] + 1.0
    dinv = lax.rsqrt(deg)
    hn_ref[...] = jnp.dot(x_ref[...], w_ref[...],
                          preferred_element_type=jnp.float32) * dinv
    dinv_ref[...] = jnp.broadcast_to(dinv, dinv_ref.shape)


_scale_mm = pl.pallas_call(
    _scale_mm_body,
    grid=(N // R,),
    in_specs=[
        pl.BlockSpec((R, D), lambda i: (i, 0)),
        pl.BlockSpec((D, D), lambda i: (0, 0)),
        pl.BlockSpec((2, R, D), lambda i: (0, i, 0)),
    ],
    out_specs=[
        pl.BlockSpec((R, D), lambda i: (i, 0)),
        pl.BlockSpec((R, 16), lambda i: (i, 0)),
    ],
    out_shape=[
        jax.ShapeDtypeStruct((N, D), jnp.float32),
        jax.ShapeDtypeStruct((N, 16), jnp.float32),
    ],
)


def _mid_body(acc_ref, hn_ref, dinv_ref, b_ref, w_ref, out_ref):
    dinv = dinv_ref[:, 0:1]
    h = jnp.maximum(
        (acc_ref[0] + acc_ref[1] + hn_ref[...]) * dinv + b_ref[...], 0.0)
    out_ref[...] = jnp.dot(h, w_ref[...],
                           preferred_element_type=jnp.float32) * dinv


_mid_call = pl.pallas_call(
    _mid_body,
    grid=(N // R,),
    in_specs=[
        pl.BlockSpec((2, R, D), lambda i: (0, i, 0)),
        pl.BlockSpec((R, D), lambda i: (i, 0)),
        pl.BlockSpec((R, 16), lambda i: (i, 0)),
        pl.BlockSpec((1, D), lambda i: (0, 0)),
        pl.BlockSpec((D, D), lambda i: (0, 0)),
    ],
    out_specs=pl.BlockSpec((R, D), lambda i: (i, 0)),
    out_shape=jax.ShapeDtypeStruct((N, D), jnp.float32),
)


def _last_body(acc_ref, hn_ref, dinv_ref, b_ref, w_ref, ob_ref, out_ref):
    dinv = dinv_ref[:, 0:1]
    h = jnp.maximum(
        (acc_ref[0] + acc_ref[1] + hn_ref[...]) * dinv + b_ref[...], 0.0)
    out_ref[...] = jnp.dot(h, w_ref[...],
                           preferred_element_type=jnp.float32) + ob_ref[...]


_last_call = pl.pallas_call(
    _last_body,
    grid=(N // R,),
    in_specs=[
        pl.BlockSpec((2, R, D), lambda i: (0, i, 0)),
        pl.BlockSpec((R, D), lambda i: (i, 0)),
        pl.BlockSpec((R, 16), lambda i: (i, 0)),
        pl.BlockSpec((1, D), lambda i: (0, 0)),
        pl.BlockSpec((D, G4), lambda i: (0, 0)),
        pl.BlockSpec((1, G4), lambda i: (0, 0)),
    ],
    out_specs=pl.BlockSpec((R, G4), lambda i: (i, 0)),
    out_shape=jax.ShapeDtypeStruct((N, G4), jnp.float32),
)


def _lstm_body(xp_ref, whh_ref, wp_ref, bp_ref, out_ref, h_s, c_s):
    t = pl.program_id(0)

    @pl.when(t == 0)
    def _():
        h_s[...] = jnp.zeros_like(h_s)
        c_s[...] = jnp.zeros_like(c_s)

    gates = xp_ref[0] + jnp.dot(h_s[...], whh_ref[...],
                                      preferred_element_type=jnp.float32)
    i = jax.nn.sigmoid(gates[:, 0:D])
    f = jax.nn.sigmoid(gates[:, D:2 * D])
    g = jnp.tanh(gates[:, 2 * D:3 * D])
    o = jax.nn.sigmoid(gates[:, 3 * D:4 * D])
    c = f * c_s[...] + i * g
    h = o * jnp.tanh(c)
    h_s[...] = h
    c_s[...] = c

    @pl.when(t == T - 1)
    def _():
        out_ref[...] = jnp.dot(h, wp_ref[...],
                               preferred_element_type=jnp.float32) + bp_ref[...]


_lstm_call = pl.pallas_call(
    _lstm_body,
    grid=(T,),
    in_specs=[
        pl.BlockSpec((1, B, G4), lambda t: (t, 0, 0)),
        pl.BlockSpec((D, G4), lambda t: (0, 0)),
        pl.BlockSpec((D, D), lambda t: (0, 0)),
        pl.BlockSpec((1, D), lambda t: (0, 0)),
    ],
    out_specs=pl.BlockSpec((B, D), lambda t: (0, 0)),
    out_shape=jax.ShapeDtypeStruct((B, D), jnp.float32),
    scratch_shapes=[
        pltpu.VMEM((B, D), jnp.float32),
        pltpu.VMEM((B, D), jnp.float32),
    ],
)


def kernel(x, edge_index, batch_size, W1, b1, W2, b2,
           W_ih, W_hh, b_ih, b_hh, Wp, bp):
    pad = E_PAD - E
    pidx = jnp.arange(pad, dtype=jnp.int32)
    src = jnp.concatenate([edge_index[0], (pidx * 13) % N])
    dst = jnp.concatenate([edge_index[1], N + (pidx % (N_PAD - N))])
    srcg = src.reshape(E_PAD // K, K)
    dstg = dst.reshape(E_PAD // K, K)

    degp = _deg_call(dstg)
    hn1, dinv = _scale_mm(x, W1, degp)

    acc1 = _edge_call(hn1, srcg, dstg)
    hn2 = _mid_call(acc1, hn1, dinv, b1.reshape(1, D), W2)
    acc2 = _edge_call(hn2, srcg, dstg)
    xproj = _last_call(acc2, hn2, dinv, b2.reshape(1, D), W_ih.T,
                       (b_ih + b_hh).reshape(1, G4))
    xtm = xproj.reshape(B, T, G4).transpose(1, 0, 2)
    out = _lstm_call(xtm, W_hh.T, Wp.T, bp.reshape(1, D))
    return out


# speedup vs baseline: 2.3584x; 1.0031x over previous
import functools

import jax
import jax.numpy as jnp
from jax import lax
from jax.experimental import pallas as pl
from jax.experimental.pallas import tpu as pltpu
from jax.experimental.pallas import tpu_sc as plsc

N = 10000
E = 320000
D = 128
T = 100
B = 100
G4 = 512

NC = 2
NS = 16
K = 128
NCHUNK = 80
EPT = NCHUNK * K
EPC = EPT * NS
E_PAD = EPC * NC
N_PAD = 10240
RPT = N_PAD // NS
ZR = 128

_mesh = functools.partial(
    plsc.VectorSubcoreMesh,
    core_axis_name="c", subcore_axis_name="s", num_cores=NC, num_subcores=NS,
)


def _deg_body(dst_hbm, out_hbm, dst_v, ones_v, acc_sh, sem):
    cid = lax.axis_index("c")
    sid = lax.axis_index("s")
    rowbase = cid * (EPC // K) + sid * NCHUNK
    pltpu.sync_copy(dst_hbm.at[pl.ds(rowbase, NCHUNK)], dst_v)

    z16 = jnp.zeros((16,), jnp.float32)
    o16 = jnp.full((16,), 1.0, jnp.float32)

    @pl.loop(0, K)
    def _(r):
        for j in range(D // 16):
            ones_v[r, pl.ds(j * 16, 16)] = z16

    for t in range(RPT // K):
        pltpu.sync_copy(ones_v, acc_sh.at[pl.ds(sid * RPT + t * K, K)])

    @pl.loop(0, K)
    def _(r):
        for j in range(D // 16):
            ones_v[r, pl.ds(j * 16, 16)] = o16

    plsc.subcore_barrier()

    @pl.loop(0, NCHUNK)
    def _(j):
        pltpu.sync_copy(ones_v, acc_sh.at[dst_v.at[j]], add=True)

    plsc.subcore_barrier()
    pltpu.sync_copy(acc_sh.at[pl.ds(sid * RPT, RPT)],
                    out_hbm.at[cid, pl.ds(sid * RPT, RPT)])


_deg_call = pl.kernel(
    _deg_body,
    out_type=jax.ShapeDtypeStruct((NC, N_PAD, D), jnp.float32),
    mesh=_mesh(),
    scratch_types=[
        pltpu.VMEM((NCHUNK, K), jnp.int32),
        pltpu.VMEM((K, D), jnp.float32),
        pltpu.VMEM_SHARED((N_PAD, D), jnp.float32),
        pltpu.SemaphoreType.DMA,
    ],
)


HC = 40


def _edge_body(hn_hbm, src_hbm, dst_hbm, out_hbm,
               src_v, dst_v, rows0_v, rows1_v, acc_sh, sem0, sem1):
    cid = lax.axis_index("c")
    sid = lax.axis_index("s")
    base = cid * (EPC // K) + sid * NCHUNK

    z16 = jnp.zeros((16,), jnp.float32)

    @pl.loop(0, K)
    def _(r):
        for j in range(D // 16):
            rows0_v[r, pl.ds(j * 16, 16)] = z16

    for t in range(RPT // K):
        pltpu.sync_copy(rows0_v, acc_sh.at[pl.ds(sid * RPT + t * K, K)])
    plsc.subcore_barrier()

    for p in range(NCHUNK // HC):
        pltpu.sync_copy(src_hbm.at[pl.ds(base + p * HC, HC)], src_v)
        pltpu.sync_copy(dst_hbm.at[pl.ds(base + p * HC, HC)], dst_v)
        pltpu.async_copy(hn_hbm.at[src_v.at[0]], rows0_v, sem0)

        @pl.loop(0, HC, step=2)
        def _(j):
            pltpu.async_copy(hn_hbm.at[src_v.at[j + 1]], rows1_v, sem1)
            pltpu.make_async_copy(
                hn_hbm.at[src_v.at[j]], rows0_v, sem0).wait()
            pltpu.sync_copy(rows0_v, acc_sh.at[dst_v.at[j]], add=True)

            @pl.when(j + 2 < HC)
            def _():
                pltpu.async_copy(hn_hbm.at[src_v.at[j + 2]], rows0_v, sem0)

            pltpu.make_async_copy(
                hn_hbm.at[src_v.at[j + 1]], rows1_v, sem1).wait()
            pltpu.sync_copy(rows1_v, acc_sh.at[dst_v.at[j + 1]], add=True)

    plsc.subcore_barrier()
    pltpu.sync_copy(acc_sh.at[pl.ds(sid * RPT, RPT)],
                    out_hbm.at[cid, pl.ds(sid * RPT, RPT)])


_edge_call = pl.kernel(
    _edge_body,
    out_type=jax.ShapeDtypeStruct((NC, N_PAD, D), jnp.float32),
    mesh=_mesh(),
    scratch_types=[
        pltpu.VMEM((HC, K), jnp.int32),
        pltpu.VMEM((HC, K), jnp.int32),
        pltpu.VMEM((K, D), jnp.float32),
        pltpu.VMEM((K, D), jnp.float32),
        pltpu.VMEM_SHARED((N_PAD, D), jnp.float32),
        pltpu.SemaphoreType.DMA,
        pltpu.SemaphoreType.DMA,
    ],
)


R = 2000


def _scale_mm_body(x_ref, w_ref, degp_ref, hn_ref, dinv_ref):
    deg = degp_ref[0, :, 0:1] + degp_ref[1, :, 0:1] + 1.0
    dinv = lax.rsqrt(deg)
    hn_ref[...] = jnp.dot(x_ref[...], w_ref[...],
                          preferred_element_type=jnp.float32) * dinv
    dinv_ref[...] = jnp.broadcast_to(dinv, dinv_ref.shape)


_scale_mm = pl.pallas_call(
    _scale_mm_body,
    grid=(N // R,),
    in_specs=[
        pl.BlockSpec((R, D), lambda i: (i, 0)),
        pl.BlockSpec((D, D), lambda i: (0, 0)),
        pl.BlockSpec((2, R, D), lambda i: (0, i, 0)),
    ],
    out_specs=[
        pl.BlockSpec((R, D), lambda i: (i, 0)),
        pl.BlockSpec((R, 16), lambda i: (i, 0)),
    ],
    out_shape=[
        jax.ShapeDtypeStruct((N, D), jnp.float32),
        jax.ShapeDtypeStruct((N, 16), jnp.float32),
    ],
)


def _mid_body(acc_ref, hn_ref, dinv_ref, b_ref, w_ref, out_ref):
    dinv = dinv_ref[:, 0:1]
    h = jnp.maximum(
        (acc_ref[0] + acc_ref[1] + hn_ref[...]) * dinv + b_ref[...], 0.0)
    out_ref[...] = jnp.dot(h, w_ref[...],
                           preferred_element_type=jnp.float32) * dinv


_mid_call = pl.pallas_call(
    _mid_body,
    grid=(N // R,),
    in_specs=[
        pl.BlockSpec((2, R, D), lambda i: (0, i, 0)),
        pl.BlockSpec((R, D), lambda i: (i, 0)),
        pl.BlockSpec((R, 16), lambda i: (i, 0)),
        pl.BlockSpec((1, D), lambda i: (0, 0)),
        pl.BlockSpec((D, D), lambda i: (0, 0)),
    ],
    out_specs=pl.BlockSpec((R, D), lambda i: (i, 0)),
    out_shape=jax.ShapeDtypeStruct((N, D), jnp.float32),
)


def _last_body(acc_ref, hn_ref, dinv_ref, b_ref, w_ref, ob_ref, out_ref):
    dinv = dinv_ref[:, 0:1]
    h = jnp.maximum(
        (acc_ref[0] + acc_ref[1] + hn_ref[...]) * dinv + b_ref[...], 0.0)
    out_ref[...] = jnp.dot(h, w_ref[...],
                           preferred_element_type=jnp.float32) + ob_ref[...]


_last_call = pl.pallas_call(
    _last_body,
    grid=(N // R,),
    in_specs=[
        pl.BlockSpec((2, R, D), lambda i: (0, i, 0)),
        pl.BlockSpec((R, D), lambda i: (i, 0)),
        pl.BlockSpec((R, 16), lambda i: (i, 0)),
        pl.BlockSpec((1, D), lambda i: (0, 0)),
        pl.BlockSpec((D, G4), lambda i: (0, 0)),
        pl.BlockSpec((1, G4), lambda i: (0, 0)),
    ],
    out_specs=pl.BlockSpec((R, G4), lambda i: (i, 0)),
    out_shape=jax.ShapeDtypeStruct((N, G4), jnp.float32),
)


def _lstm_body(xp_ref, whh_ref, wp_ref, bp_ref, out_ref, h_s, c_s):
    t = pl.program_id(0)

    @pl.when(t == 0)
    def _():
        h_s[...] = jnp.zeros_like(h_s)
        c_s[...] = jnp.zeros_like(c_s)

    gates = xp_ref[0] + jnp.dot(h_s[...], whh_ref[...],
                                      preferred_element_type=jnp.float32)
    i = jax.nn.sigmoid(gates[:, 0:D])
    f = jax.nn.sigmoid(gates[:, D:2 * D])
    g = jnp.tanh(gates[:, 2 * D:3 * D])
    o = jax.nn.sigmoid(gates[:, 3 * D:4 * D])
    c = f * c_s[...] + i * g
    h = o * jnp.tanh(c)
    h_s[...] = h
    c_s[...] = c

    @pl.when(t == T - 1)
    def _():
        out_ref[...] = jnp.dot(h, wp_ref[...],
                               preferred_element_type=jnp.float32) + bp_ref[...]


_lstm_call = pl.pallas_call(
    _lstm_body,
    grid=(T,),
    in_specs=[
        pl.BlockSpec((1, B, G4), lambda t: (t, 0, 0)),
        pl.BlockSpec((D, G4), lambda t: (0, 0)),
        pl.BlockSpec((D, D), lambda t: (0, 0)),
        pl.BlockSpec((1, D), lambda t: (0, 0)),
    ],
    out_specs=pl.BlockSpec((B, D), lambda t: (0, 0)),
    out_shape=jax.ShapeDtypeStruct((B, D), jnp.float32),
    scratch_shapes=[
        pltpu.VMEM((B, D), jnp.float32),
        pltpu.VMEM((B, D), jnp.float32),
    ],
)


def kernel(x, edge_index, batch_size, W1, b1, W2, b2,
           W_ih, W_hh, b_ih, b_hh, Wp, bp):
    pad = E_PAD - E
    pidx = jnp.arange(pad, dtype=jnp.int32)
    src = jnp.concatenate([edge_index[0], (pidx * 13) % N])
    dst = jnp.concatenate([edge_index[1], N + (pidx % (N_PAD - N))])
    srcg = src.reshape(E_PAD // K, K)
    dstg = dst.reshape(E_PAD // K, K)

    degp = _deg_call(dstg)
    hn1, dinv = _scale_mm(x, W1, degp)

    acc1 = _edge_call(hn1, srcg, dstg)
    hn2 = _mid_call(acc1, hn1, dinv, b1.reshape(1, D), W2)
    acc2 = _edge_call(hn2, srcg, dstg)
    xproj = _last_call(acc2, hn2, dinv, b2.reshape(1, D), W_ih.T,
                       (b_ih + b_hh).reshape(1, G4))
    xtm = xproj.reshape(B, T, G4).transpose(1, 0, 2)
    out = _lstm_call(xtm, W_hh.T, Wp.T, bp.reshape(1, D))
    return out


# speedup vs baseline: 2.5318x; 1.0735x over previous
import functools

import jax
import jax.numpy as jnp
from jax import lax
from jax.experimental import pallas as pl
from jax.experimental.pallas import tpu as pltpu
from jax.experimental.pallas import tpu_sc as plsc

N = 10000
E = 320000
D = 128
T = 100
B = 100
G4 = 512

NC = 2
NS = 16
K = 128
NCHUNK = 80
EPT = NCHUNK * K
EPC = EPT * NS
E_PAD = EPC * NC
N_PAD = 10240
RPT = N_PAD // NS
ZR = 128

_mesh = functools.partial(
    plsc.VectorSubcoreMesh,
    core_axis_name="c", subcore_axis_name="s", num_cores=NC, num_subcores=NS,
)


def _deg_body(dst_hbm, out_hbm, dst_v, ones_v, acc_sh, sem):
    cid = lax.axis_index("c")
    sid = lax.axis_index("s")
    rowbase = cid * (EPC // K) + sid * NCHUNK
    pltpu.sync_copy(dst_hbm.at[pl.ds(rowbase, NCHUNK)], dst_v)

    z16 = jnp.zeros((16,), jnp.float32)
    o16 = jnp.full((16,), 1.0, jnp.float32)

    @pl.loop(0, K)
    def _(r):
        for j in range(D // 16):
            ones_v[r, pl.ds(j * 16, 16)] = z16

    for t in range(RPT // K):
        pltpu.sync_copy(ones_v, acc_sh.at[pl.ds(sid * RPT + t * K, K)])

    @pl.loop(0, K)
    def _(r):
        for j in range(D // 16):
            ones_v[r, pl.ds(j * 16, 16)] = o16

    plsc.subcore_barrier()

    @pl.loop(0, NCHUNK)
    def _(j):
        pltpu.sync_copy(ones_v, acc_sh.at[dst_v.at[j]], add=True)

    plsc.subcore_barrier()
    pltpu.sync_copy(acc_sh.at[pl.ds(sid * RPT, RPT)],
                    out_hbm.at[cid, pl.ds(sid * RPT, RPT)])


_deg_call = pl.kernel(
    _deg_body,
    out_type=jax.ShapeDtypeStruct((NC, N_PAD, D), jnp.float32),
    mesh=_mesh(),
    scratch_types=[
        pltpu.VMEM((NCHUNK, K), jnp.int32),
        pltpu.VMEM((K, D), jnp.float32),
        pltpu.VMEM_SHARED((N_PAD, D), jnp.float32),
        pltpu.SemaphoreType.DMA,
    ],
)


HC = 40


def _edge_body(hn_hbm, src_hbm, dst_hbm, out_hbm,
               src_v, dst_v, rows0_v, rows1_v, acc_sh, sem0, sem1):
    cid = lax.axis_index("c")
    sid = lax.axis_index("s")
    base = cid * (EPC // K) + sid * NCHUNK

    z16 = jnp.zeros((16,), jnp.float32)

    @pl.loop(0, K)
    def _(r):
        for j in range(D // 16):
            rows0_v[r, pl.ds(j * 16, 16)] = z16

    for t in range(RPT // K):
        pltpu.sync_copy(rows0_v, acc_sh.at[pl.ds(sid * RPT + t * K, K)])
    plsc.subcore_barrier()

    for p in range(NCHUNK // HC):
        pltpu.sync_copy(src_hbm.at[pl.ds(base + p * HC, HC)], src_v)
        pltpu.sync_copy(dst_hbm.at[pl.ds(base + p * HC, HC)], dst_v)
        pltpu.async_copy(hn_hbm.at[src_v.at[0]], rows0_v, sem0)

        @pl.loop(0, HC, step=2)
        def _(j):
            pltpu.async_copy(hn_hbm.at[src_v.at[j + 1]], rows1_v, sem1)
            pltpu.make_async_copy(
                hn_hbm.at[src_v.at[j]], rows0_v, sem0).wait()
            pltpu.sync_copy(rows0_v, acc_sh.at[dst_v.at[j]], add=True)

            @pl.when(j + 2 < HC)
            def _():
                pltpu.async_copy(hn_hbm.at[src_v.at[j + 2]], rows0_v, sem0)

            pltpu.make_async_copy(
                hn_hbm.at[src_v.at[j + 1]], rows1_v, sem1).wait()
            pltpu.sync_copy(rows1_v, acc_sh.at[dst_v.at[j + 1]], add=True)

    plsc.subcore_barrier()
    pltpu.sync_copy(acc_sh.at[pl.ds(sid * RPT, RPT)],
                    out_hbm.at[cid, pl.ds(sid * RPT, RPT)])


_edge_call = pl.kernel(
    _edge_body,
    out_type=jax.ShapeDtypeStruct((NC, N_PAD, D), jnp.float32),
    mesh=_mesh(),
    scratch_types=[
        pltpu.VMEM((HC, K), jnp.int32),
        pltpu.VMEM((HC, K), jnp.int32),
        pltpu.VMEM((K, D), jnp.float32),
        pltpu.VMEM((K, D), jnp.float32),
        pltpu.VMEM_SHARED((N_PAD, D), jnp.float32),
        pltpu.SemaphoreType.DMA,
        pltpu.SemaphoreType.DMA,
    ],
)


R = 2000


def _scale_mm_body(x_ref, w_ref, degp_ref, hn_ref, dinv_ref):
    deg = degp_ref[0, :, 0:1] + degp_ref[1, :, 0:1] + 1.0
    dinv = lax.rsqrt(deg)
    hn_ref[...] = jnp.dot(x_ref[...], w_ref[...],
                          preferred_element_type=jnp.float32) * dinv
    dinv_ref[...] = jnp.broadcast_to(dinv, dinv_ref.shape)


_scale_mm = pl.pallas_call(
    _scale_mm_body,
    grid=(N // R,),
    in_specs=[
        pl.BlockSpec((R, D), lambda i: (i, 0)),
        pl.BlockSpec((D, D), lambda i: (0, 0)),
        pl.BlockSpec((2, R, D), lambda i: (0, i, 0)),
    ],
    out_specs=[
        pl.BlockSpec((R, D), lambda i: (i, 0)),
        pl.BlockSpec((R, 16), lambda i: (i, 0)),
    ],
    out_shape=[
        jax.ShapeDtypeStruct((N, D), jnp.float32),
        jax.ShapeDtypeStruct((N, 16), jnp.float32),
    ],
)


def _mid_body(acc_ref, hn_ref, dinv_ref, b_ref, w_ref, out_ref):
    dinv = dinv_ref[:, 0:1]
    h = jnp.maximum(
        (acc_ref[0] + acc_ref[1] + hn_ref[...]) * dinv + b_ref[...], 0.0)
    out_ref[...] = jnp.dot(h, w_ref[...],
                           preferred_element_type=jnp.float32) * dinv


_mid_call = pl.pallas_call(
    _mid_body,
    grid=(N // R,),
    in_specs=[
        pl.BlockSpec((2, R, D), lambda i: (0, i, 0)),
        pl.BlockSpec((R, D), lambda i: (i, 0)),
        pl.BlockSpec((R, 16), lambda i: (i, 0)),
        pl.BlockSpec((1, D), lambda i: (0, 0)),
        pl.BlockSpec((D, D), lambda i: (0, 0)),
    ],
    out_specs=pl.BlockSpec((R, D), lambda i: (i, 0)),
    out_shape=jax.ShapeDtypeStruct((N, D), jnp.float32),
)


def _last_body(acc_ref, hn_ref, dinv_ref, b_ref, w_ref, ob_ref, out_ref):
    dinv = dinv_ref[:, 0:1]
    h = jnp.maximum(
        (acc_ref[0] + acc_ref[1] + hn_ref[...]) * dinv + b_ref[...], 0.0)
    out_ref[...] = jnp.dot(h, w_ref[...],
                           preferred_element_type=jnp.float32) + ob_ref[...]


_last_call = pl.pallas_call(
    _last_body,
    grid=(N // R,),
    in_specs=[
        pl.BlockSpec((2, R, D), lambda i: (0, i, 0)),
        pl.BlockSpec((R, D), lambda i: (i, 0)),
        pl.BlockSpec((R, 16), lambda i: (i, 0)),
        pl.BlockSpec((1, D), lambda i: (0, 0)),
        pl.BlockSpec((D, G4), lambda i: (0, 0)),
        pl.BlockSpec((1, G4), lambda i: (0, 0)),
    ],
    out_specs=pl.BlockSpec((R, G4), lambda i: (i, 0)),
    out_shape=jax.ShapeDtypeStruct((N, G4), jnp.float32),
)


def _lstm_body(xp_ref, whh_ref, wp_ref, bp_ref, out_ref):
    whh = whh_ref[...]

    def step(t, hc):
        h, c = hc
        gates = xp_ref[t] + jnp.dot(h, whh, preferred_element_type=jnp.float32)
        i = jax.nn.sigmoid(gates[:, 0:D])
        f = jax.nn.sigmoid(gates[:, D:2 * D])
        g = jnp.tanh(gates[:, 2 * D:3 * D])
        o = jax.nn.sigmoid(gates[:, 3 * D:4 * D])
        c = f * c + i * g
        h = o * jnp.tanh(c)
        return (h, c)

    h0 = jnp.zeros((B, D), jnp.float32)
    c0 = jnp.zeros((B, D), jnp.float32)
    h, c = lax.fori_loop(0, T, step, (h0, c0))
    out_ref[...] = jnp.dot(h, wp_ref[...],
                           preferred_element_type=jnp.float32) + bp_ref[...]


_lstm_call = pl.pallas_call(
    _lstm_body,
    out_shape=jax.ShapeDtypeStruct((B, D), jnp.float32),
)


def kernel(x, edge_index, batch_size, W1, b1, W2, b2,
           W_ih, W_hh, b_ih, b_hh, Wp, bp):
    pad = E_PAD - E
    pidx = jnp.arange(pad, dtype=jnp.int32)
    src = jnp.concatenate([edge_index[0], (pidx * 13) % N])
    dst = jnp.concatenate([edge_index[1], N + (pidx % (N_PAD - N))])
    srcg = src.reshape(E_PAD // K, K)
    dstg = dst.reshape(E_PAD // K, K)

    degp = _deg_call(dstg)
    hn1, dinv = _scale_mm(x, W1, degp)

    acc1 = _edge_call(hn1, srcg, dstg)
    hn2 = _mid_call(acc1, hn1, dinv, b1.reshape(1, D), W2)
    acc2 = _edge_call(hn2, srcg, dstg)
    xproj = _last_call(acc2, hn2, dinv, b2.reshape(1, D), W_ih.T,
                       (b_ih + b_hh).reshape(1, G4))
    xtm = xproj.reshape(B, T, G4).transpose(1, 0, 2)
    out = _lstm_call(xtm, W_hh.T, Wp.T, bp.reshape(1, D))
    return out


# speedup vs baseline: 2.7829x; 1.0992x over previous
import functools

import jax
import jax.numpy as jnp
from jax import lax
from jax.experimental import pallas as pl
from jax.experimental.pallas import tpu as pltpu
from jax.experimental.pallas import tpu_sc as plsc

N = 10000
E = 320000
D = 128
T = 100
B = 100
G4 = 512

NC = 2
NS = 16
K = 128
NCHUNK = 80
EPT = NCHUNK * K
EPC = EPT * NS
E_PAD = EPC * NC
N_PAD = 10240
RPT = N_PAD // NS
ZR = 128

_mesh = functools.partial(
    plsc.VectorSubcoreMesh,
    core_axis_name="c", subcore_axis_name="s", num_cores=NC, num_subcores=NS,
)


def _deg_body(dst_hbm, out_hbm, dst_v, ones_v, acc_sh, sem):
    cid = lax.axis_index("c")
    sid = lax.axis_index("s")
    rowbase = cid * (EPC // K) + sid * NCHUNK
    pltpu.sync_copy(dst_hbm.at[pl.ds(rowbase, NCHUNK)], dst_v)

    z16 = jnp.zeros((16,), jnp.float32)
    o16 = jnp.full((16,), 1.0, jnp.float32)

    @pl.loop(0, K)
    def _(r):
        for j in range(D // 16):
            ones_v[r, pl.ds(j * 16, 16)] = z16

    for t in range(RPT // K):
        pltpu.sync_copy(ones_v, acc_sh.at[pl.ds(sid * RPT + t * K, K)])

    @pl.loop(0, K)
    def _(r):
        for j in range(D // 16):
            ones_v[r, pl.ds(j * 16, 16)] = o16

    plsc.subcore_barrier()

    @pl.loop(0, NCHUNK)
    def _(j):
        pltpu.sync_copy(ones_v, acc_sh.at[dst_v.at[j]], add=True)

    plsc.subcore_barrier()
    pltpu.sync_copy(acc_sh.at[pl.ds(sid * RPT, RPT)],
                    out_hbm.at[cid, pl.ds(sid * RPT, RPT)])


_deg_call = pl.kernel(
    _deg_body,
    out_type=jax.ShapeDtypeStruct((NC, N_PAD, D), jnp.float32),
    mesh=_mesh(),
    scratch_types=[
        pltpu.VMEM((NCHUNK, K), jnp.int32),
        pltpu.VMEM((K, D), jnp.float32),
        pltpu.VMEM_SHARED((N_PAD, D), jnp.float32),
        pltpu.SemaphoreType.DMA,
    ],
)


HC = 40


def _edge_body(hn_hbm, src_hbm, dst_hbm, out_hbm,
               src_v, dst_v, rows0_v, rows1_v, acc_sh, sem0, sem1):
    cid = lax.axis_index("c")
    sid = lax.axis_index("s")
    base = cid * (EPC // K) + sid * NCHUNK

    z16 = jnp.zeros((16,), jnp.float32)

    @pl.loop(0, K)
    def _(r):
        for j in range(D // 16):
            rows0_v[r, pl.ds(j * 16, 16)] = z16

    for t in range(RPT // K):
        pltpu.sync_copy(rows0_v, acc_sh.at[pl.ds(sid * RPT + t * K, K)])
    plsc.subcore_barrier()

    for p in range(NCHUNK // HC):
        pltpu.sync_copy(src_hbm.at[pl.ds(base + p * HC, HC)], src_v)
        pltpu.sync_copy(dst_hbm.at[pl.ds(base + p * HC, HC)], dst_v)
        pltpu.async_copy(hn_hbm.at[src_v.at[0]], rows0_v, sem0)

        @pl.loop(0, HC, step=2)
        def _(j):
            pltpu.async_copy(hn_hbm.at[src_v.at[j + 1]], rows1_v, sem1)
            pltpu.make_async_copy(
                hn_hbm.at[src_v.at[j]], rows0_v, sem0).wait()
            pltpu.sync_copy(rows0_v, acc_sh.at[dst_v.at[j]], add=True)

            @pl.when(j + 2 < HC)
            def _():
                pltpu.async_copy(hn_hbm.at[src_v.at[j + 2]], rows0_v, sem0)

            pltpu.make_async_copy(
                hn_hbm.at[src_v.at[j + 1]], rows1_v, sem1).wait()
            pltpu.sync_copy(rows1_v, acc_sh.at[dst_v.at[j + 1]], add=True)

    plsc.subcore_barrier()
    pltpu.sync_copy(acc_sh.at[pl.ds(sid * RPT, RPT)],
                    out_hbm.at[cid, pl.ds(sid * RPT, RPT)])


_edge_call = pl.kernel(
    _edge_body,
    out_type=jax.ShapeDtypeStruct((NC, N_PAD, D), jnp.float32),
    mesh=_mesh(),
    scratch_types=[
        pltpu.VMEM((HC, K), jnp.int32),
        pltpu.VMEM((HC, K), jnp.int32),
        pltpu.VMEM((K, D), jnp.float32),
        pltpu.VMEM((K, D), jnp.float32),
        pltpu.VMEM_SHARED((N_PAD, D), jnp.float32),
        pltpu.SemaphoreType.DMA,
        pltpu.SemaphoreType.DMA,
    ],
)


R = 2000


def _scale_mm_body(x_ref, w_ref, degp_ref, hn_ref, dinv_ref):
    deg = degp_ref[0, :, 0:1] + degp_ref[1, :, 0:1] + 1.0
    dinv = lax.rsqrt(deg)
    hn_ref[...] = jnp.dot(x_ref[...], w_ref[...],
                          preferred_element_type=jnp.float32) * dinv
    dinv_ref[...] = jnp.broadcast_to(dinv, dinv_ref.shape)


_scale_mm = pl.pallas_call(
    _scale_mm_body,
    grid=(N // R,),
    in_specs=[
        pl.BlockSpec((R, D), lambda i: (i, 0)),
        pl.BlockSpec((D, D), lambda i: (0, 0)),
        pl.BlockSpec((2, R, D), lambda i: (0, i, 0)),
    ],
    out_specs=[
        pl.BlockSpec((R, D), lambda i: (i, 0)),
        pl.BlockSpec((R, 16), lambda i: (i, 0)),
    ],
    out_shape=[
        jax.ShapeDtypeStruct((N, D), jnp.float32),
        jax.ShapeDtypeStruct((N, 16), jnp.float32),
    ],
)


def _mid_body(acc_ref, hn_ref, dinv_ref, b_ref, w_ref, out_ref):
    dinv = dinv_ref[:, 0:1]
    h = jnp.maximum(
        (acc_ref[0] + acc_ref[1] + hn_ref[...]) * dinv + b_ref[...], 0.0)
    out_ref[...] = jnp.dot(h, w_ref[...],
                           preferred_element_type=jnp.float32) * dinv


_mid_call = pl.pallas_call(
    _mid_body,
    grid=(N // R,),
    in_specs=[
        pl.BlockSpec((2, R, D), lambda i: (0, i, 0)),
        pl.BlockSpec((R, D), lambda i: (i, 0)),
        pl.BlockSpec((R, 16), lambda i: (i, 0)),
        pl.BlockSpec((1, D), lambda i: (0, 0)),
        pl.BlockSpec((D, D), lambda i: (0, 0)),
    ],
    out_specs=pl.BlockSpec((R, D), lambda i: (i, 0)),
    out_shape=jax.ShapeDtypeStruct((N, D), jnp.float32),
)


def _last_body(acc_ref, hn_ref, dinv_ref, b_ref, out_ref):
    dinv = dinv_ref[:, 0:1]
    out_ref[...] = jnp.maximum(
        (acc_ref[0] + acc_ref[1] + hn_ref[...]) * dinv + b_ref[...], 0.0)


_last_call = pl.pallas_call(
    _last_body,
    grid=(N // R,),
    in_specs=[
        pl.BlockSpec((2, R, D), lambda i: (0, i, 0)),
        pl.BlockSpec((R, D), lambda i: (i, 0)),
        pl.BlockSpec((R, 16), lambda i: (i, 0)),
        pl.BlockSpec((1, D), lambda i: (0, 0)),
    ],
    out_specs=pl.BlockSpec((R, D), lambda i: (i, 0)),
    out_shape=jax.ShapeDtypeStruct((N, D), jnp.float32),
)


def _lstm_body(xp_ref, wih_ref, whh_ref, bio_ref, wp_ref, bp_ref, out_ref):
    wih = wih_ref[...]
    whh = whh_ref[...]
    bio = bio_ref[...]

    def step(t, hc):
        h, c = hc
        xt = xp_ref[:, t, :]
        gates = (jnp.dot(xt, wih, preferred_element_type=jnp.float32) + bio
                 + jnp.dot(h, whh, preferred_element_type=jnp.float32))
        i = jax.nn.sigmoid(gates[:, 0:D])
        f = jax.nn.sigmoid(gates[:, D:2 * D])
        g = jnp.tanh(gates[:, 2 * D:3 * D])
        o = jax.nn.sigmoid(gates[:, 3 * D:4 * D])
        c = f * c + i * g
        h = o * jnp.tanh(c)
        return (h, c)

    h0 = jnp.zeros((B, D), jnp.float32)
    c0 = jnp.zeros((B, D), jnp.float32)
    h, c = lax.fori_loop(0, T, step, (h0, c0))
    out_ref[...] = jnp.dot(h, wp_ref[...],
                           preferred_element_type=jnp.float32) + bp_ref[...]


_lstm_call = pl.pallas_call(
    _lstm_body,
    out_shape=jax.ShapeDtypeStruct((B, D), jnp.float32),
)


def kernel(x, edge_index, batch_size, W1, b1, W2, b2,
           W_ih, W_hh, b_ih, b_hh, Wp, bp):
    pad = E_PAD - E
    pidx = jnp.arange(pad, dtype=jnp.int32)
    src = jnp.concatenate([edge_index[0], (pidx * 13) % N])
    dst = jnp.concatenate([edge_index[1], N + (pidx % (N_PAD - N))])
    srcg = src.reshape(E_PAD // K, K)
    dstg = dst.reshape(E_PAD // K, K)

    degp = _deg_call(dstg)
    hn1, dinv = _scale_mm(x, W1, degp)

    acc1 = _edge_call(hn1, srcg, dstg)
    hn2 = _mid_call(acc1, hn1, dinv, b1.reshape(1, D), W2)
    acc2 = _edge_call(hn2, srcg, dstg)
    h2 = _last_call(acc2, hn2, dinv, b2.reshape(1, D))
    out = _lstm_call(h2.reshape(B, T, D), W_ih.T, W_hh.T,
                     (b_ih + b_hh).reshape(1, G4), Wp.T, bp.reshape(1, D))
    return out
